# baseline (speedup 1.0000x reference)
import functools

import jax
import jax.numpy as jnp
from jax import lax
from jax.experimental import pallas as pl
from jax.experimental.pallas import tpu as pltpu
from jax.experimental.pallas import tpu_sc as plsc

N = 10000
E = 320000
D = 128
O = 128
R = 8

NC = 2
NS = 16
NW = NC * NS
EPT = E // NW
CH = 80
NCH = EPT // CH
NBUF = 3
GA = 2
LANES = 16

ROWS_A = 624
ROWS_LAST = N - (NS - 1) * ROWS_A

BN = 2000
NB = N // BN
EROW = E // 128


def _dense_body(h_ref, x_ref, et_ref, src_ref, wrel_ref, wroot_ref,
                wskip_ref, bias_ref, bskip_ref, hrel_ref, base_ref, idx_ref):
    nb = pl.program_id(0)
    r = pl.program_id(1)
    hblk = h_ref[...]
    hrel_ref[...] = jnp.dot(hblk, wrel_ref[0],
                            preferred_element_type=jnp.float32)

    @pl.when(r == 0)
    def _():
        skip = lax.dot_general(x_ref[...], wskip_ref[...],
                               (((1,), (1,)), ((), ())),
                               preferred_element_type=jnp.float32)
        root = jnp.dot(hblk, wroot_ref[...],
                       preferred_element_type=jnp.float32)
        base_ref[...] = skip + root + bias_ref[...] + bskip_ref[...]

    @pl.when((r == 0) & (nb == 0))
    def _():
        idx_ref[...] = et_ref[...] * N + src_ref[...]


_dense = pl.pallas_call(
    _dense_body,
    grid=(NB, R),
    in_specs=[
        pl.BlockSpec((BN, D), lambda nb, r: (nb, 0)),
        pl.BlockSpec((BN, D), lambda nb, r: (nb, 0)),
        pl.BlockSpec((EROW, 128), lambda nb, r: (0, 0)),
        pl.BlockSpec((EROW, 128), lambda nb, r: (0, 0)),
        pl.BlockSpec((1, D, O), lambda nb, r: (r, 0, 0)),
        pl.BlockSpec((D, O), lambda nb, r: (0, 0)),
        pl.BlockSpec((O, D), lambda nb, r: (0, 0)),
        pl.BlockSpec((1, O), lambda nb, r: (0, 0)),
        pl.BlockSpec((1, O), lambda nb, r: (0, 0)),
    ],
    out_specs=[
        pl.BlockSpec((BN, O), lambda nb, r: (r * NB + nb, 0)),
        pl.BlockSpec((BN, O), lambda nb, r: (nb, 0)),
        pl.BlockSpec((EROW, 128), lambda nb, r: (0, 0)),
    ],
    out_shape=[
        jax.ShapeDtypeStruct((R * N, O), jnp.float32),
        jax.ShapeDtypeStruct((N, O), jnp.float32),
        jax.ShapeDtypeStruct((EROW, 128), jnp.int32),
    ],
)


def _sc_edges_body(idx_hbm, ew_hbm, dst_hbm, hrel_hbm, zeros_hbm,
                   out_hbm, idx_v, dstb_v, wb_v, rows_v, acc_s,
                   gsem, ssem, dsem, wsem):
    c = lax.axis_index("c")
    s = lax.axis_index("s")
    wid = c * NS + s

    @pl.when(s < NS - 1)
    def _():
        pltpu.sync_copy(zeros_hbm.at[pl.ds(s * ROWS_A, ROWS_A)],
                        acc_s.at[pl.ds(s * ROWS_A, ROWS_A)])

    @pl.when(s == NS - 1)
    def _():
        pltpu.sync_copy(zeros_hbm.at[pl.ds((NS - 1) * ROWS_A, ROWS_LAST)],
                        acc_s.at[pl.ds((NS - 1) * ROWS_A, ROWS_LAST)])

    ebase = wid * EPT
    pltpu.sync_copy(idx_hbm.at[pl.ds(ebase, EPT)], idx_v)

    plsc.subcore_barrier()

    def issue(k, b):
        pltpu.async_copy(dst_hbm.at[pl.ds(ebase + k * CH, CH)], dstb_v.at[b],
                         dsem)
        pltpu.async_copy(ew_hbm.at[pl.ds(ebase + k * CH, CH)], wb_v.at[b],
                         wsem)
        pltpu.async_copy(hrel_hbm.at[idx_v.at[pl.ds(k * CH, CH)]],
                         rows_v.at[b], gsem)

    def drain_rows(sem, b):
        pltpu.make_async_copy(zeros_hbm.at[pl.ds(0, CH)],
                              rows_v.at[b], sem).wait()

    def drain_meta(sem, ref):
        pltpu.make_async_copy(ew_hbm.at[pl.ds(0, CH)], ref, sem).wait()

    for j in range(GA):
        issue(j, j)

    def chunk(k, carry):
        b = lax.rem(k, NBUF)
        drain_rows(gsem, b)
        drain_meta(wsem, wb_v.at[b])

        def scale(g, carry2):
            wv = wb_v[b, pl.ds(g * LANES, LANES)]
            for j in range(LANES):
                e = g * LANES + j
                w = wv[j]
                for c16 in range(O // LANES):
                    sl = pl.ds(c16 * LANES, LANES)
                    rows_v[b, e, sl] = rows_v[b, e, sl] * w
            return carry2

        lax.fori_loop(0, CH // LANES, scale, 0)

        drain_meta(dsem, dstb_v.at[b])
        pltpu.async_copy(rows_v.at[b], acc_s.at[dstb_v.at[b]], ssem,
                         add=True)

        @pl.when(k + GA < NCH)
        def _():
            @pl.when(k >= 1)
            def _():
                drain_rows(ssem, b)
            issue(k + GA, lax.rem(k + GA, NBUF))

        return carry

    lax.fori_loop(0, NCH, chunk, 0)

    for j in range(NBUF):
        drain_rows(ssem, j)

    plsc.subcore_barrier()

    @pl.when(s < NS - 1)
    def _():
        pltpu.sync_copy(acc_s.at[pl.ds(s * ROWS_A, ROWS_A)],
                        out_hbm.at[c, pl.ds(s * ROWS_A, ROWS_A)])

    @pl.when(s == NS - 1)
    def _():
        pltpu.sync_copy(acc_s.at[pl.ds((NS - 1) * ROWS_A, ROWS_LAST)],
                        out_hbm.at[c, pl.ds((NS - 1) * ROWS_A, ROWS_LAST)])


_sc_edges = functools.partial(
    pl.kernel,
    mesh=plsc.VectorSubcoreMesh(core_axis_name="c", subcore_axis_name="s"),
    out_type=jax.ShapeDtypeStruct((NC, N, O), jnp.float32),
    scratch_types=[
        pltpu.VMEM((EPT,), jnp.int32),
        pltpu.VMEM((NBUF, CH), jnp.int32),
        pltpu.VMEM((NBUF, CH), jnp.float32),
        pltpu.VMEM((NBUF, CH, O), jnp.float32),
        pltpu.VMEM_SHARED((N, O), jnp.float32),
        pltpu.SemaphoreType.DMA,
        pltpu.SemaphoreType.DMA,
        pltpu.SemaphoreType.DMA,
        pltpu.SemaphoreType.DMA,
    ],
)(_sc_edges_body)


def _combine_body(p0_ref, p1_ref, base_ref, out_ref):
    out_ref[...] = p0_ref[...] + p1_ref[...] + base_ref[...]


_combine = pl.pallas_call(
    _combine_body,
    grid=(NB,),
    in_specs=[pl.BlockSpec((BN, O), lambda nb: (nb, 0))] * 3,
    out_specs=pl.BlockSpec((BN, O), lambda nb: (nb, 0)),
    out_shape=jax.ShapeDtypeStruct((N, O), jnp.float32),
)


def kernel(x, h, edge_index, edge_type, edge_weight, W_rel, W_root, bias,
           W_skip, b_skip):
    src = edge_index[0].reshape(EROW, 128)
    et = edge_type.reshape(EROW, 128)
    zeros = jnp.zeros((N, O), jnp.float32)

    hrel, base, idx = _dense(h, x, et, src, W_rel, W_root, W_skip,
                             bias.reshape(1, O), b_skip.reshape(1, O))
    partials = _sc_edges(idx.reshape(E), edge_weight, edge_index[1],
                         hrel, zeros)
    return _combine(partials[0], partials[1], base)
